# Initial kernel scaffold; baseline (speedup 1.0000x reference)
import jax
import jax.numpy as jnp
from jax.experimental import pallas as pl


def kernel(x, edge_index, W1, b1, W2, b2):
    raise NotImplementedError("write your pallas kernel here")


# speedup vs baseline: 23.0189x; 23.0189x over previous
import functools

import jax
import jax.numpy as jnp
from jax import lax
from jax.experimental import pallas as pl
from jax.experimental.pallas import tpu as pltpu
from jax.experimental.pallas import tpu_sc as plsc

_NC = 2
_NS = 16
_NW = _NC * _NS
_LANES = 16


def _sc_degree(dst2, n_nodes):
  ept = dst2.shape[1]
  mesh = plsc.VectorSubcoreMesh(core_axis_name="c", subcore_axis_name="s")

  @functools.partial(
      pl.kernel,
      out_type=jax.ShapeDtypeStruct((_NW, n_nodes), jnp.float32),
      mesh=mesh,
      compiler_params=pltpu.CompilerParams(needs_layout_passes=False),
      scratch_types=[
          pltpu.VMEM((ept,), jnp.int32),
          pltpu.VMEM((n_nodes,), jnp.float32),
      ],
  )
  def k(dst_hbm, out_hbm, dst_v, deg_v):
    c = lax.axis_index("c")
    s = lax.axis_index("s")
    wid = c * _NS + s
    pltpu.sync_copy(dst_hbm.at[wid], dst_v)

    def zero_body(i, carry):
      deg_v[pl.ds(pl.multiple_of(i * _LANES, 8), _LANES)] = jnp.zeros(
          (_LANES,), jnp.float32)
      return carry

    lax.fori_loop(0, n_nodes // _LANES, zero_body, 0)

    ones = jnp.ones((_LANES,), jnp.float32)

    def add_body(i, carry):
      idx = dst_v[pl.ds(pl.multiple_of(i * _LANES, 8), _LANES)]
      plsc.addupdate_scatter(deg_v, [idx], ones)
      return carry

    lax.fori_loop(0, ept // _LANES, add_body, 0)
    pltpu.sync_copy(deg_v, out_hbm.at[wid])

  return k(dst2)


def _sc_aggregate(g, src2, dst3, zeros, n_pad):
  d = g.shape[1]
  nb, b = dst3.shape[1], dst3.shape[2]
  rpt = n_pad // _NS
  mesh = plsc.VectorSubcoreMesh(core_axis_name="c", subcore_axis_name="s")

  @functools.partial(
      pl.kernel,
      out_type=jax.ShapeDtypeStruct((_NC, n_pad, d), jnp.float32),
      mesh=mesh,
      compiler_params=pltpu.CompilerParams(
          needs_layout_passes=False, use_tc_tiling_on_sc=False),
      scratch_types=[
          pltpu.VMEM((nb * b,), jnp.int32),
          pltpu.VMEM((nb, b), jnp.int32),
          pltpu.VMEM((b, d), jnp.float32),
          pltpu.VMEM_SHARED((n_pad, d), jnp.float32),
      ],
  )
  def k(g_hbm, src_hbm, dst_hbm, z_hbm, out_hbm, src_v, dst_v, rows_v, acc):
    c = lax.axis_index("c")
    s = lax.axis_index("s")
    wid = c * _NS + s
    pltpu.sync_copy(z_hbm.at[pl.ds(s * rpt, rpt)], acc.at[pl.ds(s * rpt, rpt)])
    pltpu.sync_copy(src_hbm.at[wid], src_v)
    pltpu.sync_copy(dst_hbm.at[wid], dst_v)
    plsc.subcore_barrier()

    def step(j, carry):
      off = pl.multiple_of(j * b, 8)
      pltpu.sync_copy(g_hbm.at[src_v.at[pl.ds(off, b)]], rows_v)
      pltpu.sync_copy(rows_v, acc.at[dst_v.at[j]], add=True)
      return carry

    lax.fori_loop(0, nb, step, 0)
    plsc.subcore_barrier()
    pltpu.sync_copy(acc.at[pl.ds(s * rpt, rpt)],
                    out_hbm.at[c, pl.ds(s * rpt, rpt)])

  return k(g, src2, dst3, zeros)


def _dinv_block(degpt_ref):
  return lax.rsqrt(1.0 + jnp.sum(degpt_ref[...], axis=1, keepdims=True))


def _tc1_body(x_ref, w_ref, degpt_ref, out_ref):
  h = jnp.dot(x_ref[...], w_ref[...], preferred_element_type=jnp.float32)
  out_ref[...] = h * _dinv_block(degpt_ref)


def _tc2_body(acca_ref, accb_ref, g1_ref, degpt_ref, b1_ref, w2_ref, out_ref):
  dinv = _dinv_block(degpt_ref)
  t = dinv * (acca_ref[...] + accb_ref[...] + g1_ref[...]) + b1_ref[...]
  t = jnp.maximum(t, 0.0)
  out_ref[...] = jnp.dot(t, w2_ref[...],
                         preferred_element_type=jnp.float32) * dinv


def _tc3_body(acca_ref, accb_ref, g2_ref, degpt_ref, b2_ref, out_ref):
  dinv = _dinv_block(degpt_ref)
  out_ref[...] = dinv * (acca_ref[...] + accb_ref[...] + g2_ref[...]) + b2_ref[...]


_BM = 1000


def _tc1(x, w1, degpt):
  n, f = x.shape
  h = w1.shape[1]
  return pl.pallas_call(
      _tc1_body,
      grid=(n // _BM,),
      in_specs=[
          pl.BlockSpec((_BM, f), lambda i: (i, 0)),
          pl.BlockSpec((f, h), lambda i: (0, 0)),
          pl.BlockSpec((_BM, _NW), lambda i: (i, 0)),
      ],
      out_specs=pl.BlockSpec((_BM, h), lambda i: (i, 0)),
      out_shape=jax.ShapeDtypeStruct((n, h), jnp.float32),
  )(x, w1, degpt)


def _tc2(acca, accb, g1, degpt, b1, w2):
  n, h = g1.shape
  co = w2.shape[1]
  return pl.pallas_call(
      _tc2_body,
      grid=(n // _BM,),
      in_specs=[
          pl.BlockSpec((_BM, h), lambda i: (i, 0)),
          pl.BlockSpec((_BM, h), lambda i: (i, 0)),
          pl.BlockSpec((_BM, h), lambda i: (i, 0)),
          pl.BlockSpec((_BM, _NW), lambda i: (i, 0)),
          pl.BlockSpec((1, h), lambda i: (0, 0)),
          pl.BlockSpec((h, co), lambda i: (0, 0)),
      ],
      out_specs=pl.BlockSpec((_BM, co), lambda i: (i, 0)),
      out_shape=jax.ShapeDtypeStruct((n, co), jnp.float32),
  )(acca, accb, g1, degpt, b1, w2)


def _tc3(acca, accb, g2, degpt, b2):
  n, co = g2.shape
  return pl.pallas_call(
      _tc3_body,
      grid=(n // _BM,),
      in_specs=[
          pl.BlockSpec((_BM, co), lambda i: (i, 0)),
          pl.BlockSpec((_BM, co), lambda i: (i, 0)),
          pl.BlockSpec((_BM, co), lambda i: (i, 0)),
          pl.BlockSpec((_BM, _NW), lambda i: (i, 0)),
          pl.BlockSpec((1, co), lambda i: (0, 0)),
      ],
      out_specs=pl.BlockSpec((_BM, co), lambda i: (i, 0)),
      out_shape=jax.ShapeDtypeStruct((n, co), jnp.float32),
  )(acca, accb, g2, degpt, b2)


def kernel(x, edge_index, W1, b1, W2, b2):
  n, f = x.shape
  e = edge_index.shape[1]
  src = edge_index[0].astype(jnp.int32)
  dst = edge_index[1].astype(jnp.int32)

  ept = e // _NW
  b = 80
  nb = ept // b
  assert ept * _NW == e and nb * b == ept and n % _NS == 0 and n % _LANES == 0

  src2 = src.reshape(_NW, ept)
  dst2 = dst.reshape(_NW, ept)
  dst3 = dst.reshape(_NW, nb, b)

  n_pad = ((n + 8 * _NS - 1) // (8 * _NS)) * (8 * _NS)

  degp = _sc_degree(dst2, n)
  degpt = degp.T

  g1 = _tc1(x, W1, degpt)
  acc1 = _sc_aggregate(
      g1, src2, dst3, jnp.zeros((n_pad, g1.shape[1]), jnp.float32), n_pad)
  g2 = _tc2(acc1[0, :n], acc1[1, :n], g1, degpt, b1.reshape(1, -1), W2)
  acc2 = _sc_aggregate(
      g2, src2, dst3, jnp.zeros((n_pad, g2.shape[1]), jnp.float32), n_pad)
  out = _tc3(acc2[0, :n], acc2[1, :n], g2, degpt, b2.reshape(1, -1))
  return out
